# Initial kernel scaffold; baseline (speedup 1.0000x reference)
import jax
import jax.numpy as jnp
from jax.experimental import pallas as pl


def kernel(x, edge_index, W1, a_src1, a_dst1, b1, W2, a_src2, a_dst2, b2, Wout, bout):
    raise NotImplementedError("write your pallas kernel here")


# speedup vs baseline: 20.4823x; 20.4823x over previous
import functools

import jax
import jax.numpy as jnp
from jax import lax
from jax.experimental import pallas as pl
from jax.experimental.pallas import tpu as pltpu
from jax.experimental.pallas import tpu_sc as plsc

N = 10000
E = 320000
D = 128
HP = 144
BM = 1000
GM = N // BM
NC = 2
NS = 16
NTILES = NC * NS
TPW = E // NTILES
EK = 80
NCHUNK = TPW // EK
NP = 10240
RPT = NP // NS


def _leaky(z):
    return jnp.where(z >= 0, z, 0.2 * z)


def _tc_embed_body(x_ref, w_ref, asv_ref, adv_ref,
                   hh_ref, as_ref, ad_ref, mx_ref):
    h = jnp.dot(x_ref[...], w_ref[...], preferred_element_type=jnp.float32)
    col = lax.broadcasted_iota(jnp.int32, h.shape, 1)
    hh_ref[...] = h + jnp.where(col == D, 1.0, 0.0)
    a_s = jnp.dot(h[:, :D], asv_ref[...], preferred_element_type=jnp.float32)
    a_d = jnp.dot(h[:, :D], adv_ref[...], preferred_element_type=jnp.float32)
    as_ref[...] = a_s
    ad_ref[...] = a_d
    mrow = jnp.concatenate([jnp.max(a_s, keepdims=True),
                            jnp.max(a_d, keepdims=True)], axis=1)

    @pl.when(pl.program_id(0) == 0)
    def _():
        mx_ref[...] = mrow

    @pl.when(pl.program_id(0) != 0)
    def _():
        mx_ref[...] = jnp.maximum(mx_ref[...], mrow)


def _tc_embed(x, wp, asv, adv):
    return pl.pallas_call(
        _tc_embed_body,
        grid=(GM,),
        in_specs=[
            pl.BlockSpec((BM, D), lambda i: (i, 0)),
            pl.BlockSpec((D, HP), lambda i: (0, 0)),
            pl.BlockSpec((D, 1), lambda i: (0, 0)),
            pl.BlockSpec((D, 1), lambda i: (0, 0)),
        ],
        out_specs=[
            pl.BlockSpec((BM, HP), lambda i: (i, 0)),
            pl.BlockSpec((BM, 1), lambda i: (i, 0)),
            pl.BlockSpec((BM, 1), lambda i: (i, 0)),
            pl.BlockSpec((1, 2), lambda i: (0, 0)),
        ],
        out_shape=[
            jax.ShapeDtypeStruct((N, HP), jnp.float32),
            jax.ShapeDtypeStruct((N, 1), jnp.float32),
            jax.ShapeDtypeStruct((N, 1), jnp.float32),
            jax.ShapeDtypeStruct((1, 2), jnp.float32),
        ],
        compiler_params=pltpu.CompilerParams(
            dimension_semantics=("arbitrary",)),
    )(x, wp, asv, adv)


def _combine(p_ref, hhp_ref, asp_ref, adp_ref, m_ref, b_ref):
    tot = p_ref[0] + p_ref[1]
    z = asp_ref[...] + adp_ref[...]
    w_self = jnp.exp(_leaky(z) - m_ref[0, 0])
    tot = tot + w_self * hhp_ref[...]
    return jnp.maximum(tot[:, :D] / (tot[:, D:D + 1] + 1e-16) + b_ref[...],
                       0.0)


def _tc_combine_body(p_ref, hhp_ref, asp_ref, adp_ref, m_ref, b_ref,
                     w_ref, asv_ref, adv_ref,
                     hh_ref, as_ref, ad_ref, mx_ref):
    xn = _combine(p_ref, hhp_ref, asp_ref, adp_ref, m_ref, b_ref)
    h = jnp.dot(xn, w_ref[...], preferred_element_type=jnp.float32)
    col = lax.broadcasted_iota(jnp.int32, h.shape, 1)
    hh_ref[...] = h + jnp.where(col == D, 1.0, 0.0)
    a_s = jnp.dot(h[:, :D], asv_ref[...], preferred_element_type=jnp.float32)
    a_d = jnp.dot(h[:, :D], adv_ref[...], preferred_element_type=jnp.float32)
    as_ref[...] = a_s
    ad_ref[...] = a_d
    mrow = jnp.concatenate([jnp.max(a_s, keepdims=True),
                            jnp.max(a_d, keepdims=True)], axis=1)

    @pl.when(pl.program_id(0) == 0)
    def _():
        mx_ref[...] = mrow

    @pl.when(pl.program_id(0) != 0)
    def _():
        mx_ref[...] = jnp.maximum(mx_ref[...], mrow)


def _tc_combine(pm, hhp, asp, adp, mscal, bias, wp, asv, adv):
    return pl.pallas_call(
        _tc_combine_body,
        grid=(GM,),
        in_specs=[
            pl.BlockSpec((2, BM, HP), lambda i: (0, i, 0)),
            pl.BlockSpec((BM, HP), lambda i: (i, 0)),
            pl.BlockSpec((BM, 1), lambda i: (i, 0)),
            pl.BlockSpec((BM, 1), lambda i: (i, 0)),
            pl.BlockSpec((1, 1), lambda i: (0, 0)),
            pl.BlockSpec((1, D), lambda i: (0, 0)),
            pl.BlockSpec((D, HP), lambda i: (0, 0)),
            pl.BlockSpec((D, 1), lambda i: (0, 0)),
            pl.BlockSpec((D, 1), lambda i: (0, 0)),
        ],
        out_specs=[
            pl.BlockSpec((BM, HP), lambda i: (i, 0)),
            pl.BlockSpec((BM, 1), lambda i: (i, 0)),
            pl.BlockSpec((BM, 1), lambda i: (i, 0)),
            pl.BlockSpec((1, 2), lambda i: (0, 0)),
        ],
        out_shape=[
            jax.ShapeDtypeStruct((N, HP), jnp.float32),
            jax.ShapeDtypeStruct((N, 1), jnp.float32),
            jax.ShapeDtypeStruct((N, 1), jnp.float32),
            jax.ShapeDtypeStruct((1, 2), jnp.float32),
        ],
        compiler_params=pltpu.CompilerParams(
            dimension_semantics=("arbitrary",)),
    )(pm, hhp, asp, adp, mscal, bias, wp, asv, adv)


def _tc_final_body(p_ref, hhp_ref, asp_ref, adp_ref, m_ref, b_ref, sum_ref):
    y = _combine(p_ref, hhp_ref, asp_ref, adp_ref, m_ref, b_ref)
    ps = jnp.sum(y, axis=0, keepdims=True)

    @pl.when(pl.program_id(0) == 0)
    def _():
        sum_ref[...] = ps

    @pl.when(pl.program_id(0) != 0)
    def _():
        sum_ref[...] = sum_ref[...] + ps


def _tc_final(pm, hhp, asp, adp, mscal, bias):
    return pl.pallas_call(
        _tc_final_body,
        grid=(GM,),
        in_specs=[
            pl.BlockSpec((2, BM, HP), lambda i: (0, i, 0)),
            pl.BlockSpec((BM, HP), lambda i: (i, 0)),
            pl.BlockSpec((BM, 1), lambda i: (i, 0)),
            pl.BlockSpec((BM, 1), lambda i: (i, 0)),
            pl.BlockSpec((1, 1), lambda i: (0, 0)),
            pl.BlockSpec((1, D), lambda i: (0, 0)),
        ],
        out_specs=[pl.BlockSpec((1, D), lambda i: (0, 0))],
        out_shape=[jax.ShapeDtypeStruct((1, D), jnp.float32)],
        compiler_params=pltpu.CompilerParams(
            dimension_semantics=("arbitrary",)),
    )(pm, hhp, asp, adp, mscal, bias)


def _sc_edge_body(src_hbm, dst_hbm, hh_hbm, as_hbm, ad_hbm, mv_hbm, zero_hbm,
                  out_hbm,
                  src_v, dst_v, rows_v, w_v, as_l, ad_l, m_l,
                  msg_sh, sem_g, sem_s):
    c = lax.axis_index("c")
    s = lax.axis_index("s")
    wid = s * NC + c

    pltpu.sync_copy(as_hbm, as_l)
    pltpu.sync_copy(ad_hbm, ad_l)
    pltpu.sync_copy(mv_hbm, m_l)

    r0 = s * RPT
    pltpu.sync_copy(zero_hbm.at[pl.ds(r0, RPT)], msg_sh.at[pl.ds(r0, RPT)])
    plsc.subcore_barrier()

    base = wid * TPW
    m16 = m_l[...]

    def chunk(ci, _):
        off = base + ci * EK
        pltpu.sync_copy(src_hbm.at[pl.ds(off, EK)], src_v)
        pltpu.sync_copy(dst_hbm.at[pl.ds(off, EK)], dst_v)
        pltpu.async_copy(hh_hbm.at[src_v], rows_v, sem_g).wait()

        def wgrp(g, _):
            sv = src_v[pl.ds(g * 16, 16)]
            dv = dst_v[pl.ds(g * 16, 16)]
            z = plsc.load_gather(as_l, [sv]) + plsc.load_gather(ad_l, [dv])
            w_v[pl.ds(g * 16, 16)] = jnp.exp(_leaky(z) - m16)
            return 0

        lax.fori_loop(0, EK // 16, wgrp, 0)

        def scale(j, _):
            wj = plsc.load_gather(w_v, [jnp.full((16,), j, jnp.int32)])
            for g in range(HP // 16):
                rows_v[j, pl.ds(g * 16, 16)] = (
                    rows_v[j, pl.ds(g * 16, 16)] * wj)
            return 0

        lax.fori_loop(0, EK, scale, 0)
        pltpu.async_copy(rows_v, msg_sh.at[dst_v], sem_s, add=True).wait()
        return 0

    lax.fori_loop(0, NCHUNK, chunk, 0)
    plsc.subcore_barrier()
    pltpu.sync_copy(msg_sh.at[pl.ds(r0, RPT)], out_hbm.at[c, pl.ds(r0, RPT)])


def _make_sc_edge():
    mesh = plsc.VectorSubcoreMesh(core_axis_name="c", subcore_axis_name="s",
                                  num_cores=NC, num_subcores=NS)

    return pl.kernel(
        _sc_edge_body,
        out_type=jax.ShapeDtypeStruct((NC, NP, HP), jnp.float32),
        mesh=mesh,
        compiler_params=pltpu.CompilerParams(needs_layout_passes=False,
                                             use_tc_tiling_on_sc=False),
        scratch_types=[
            pltpu.VMEM((EK,), jnp.int32),
            pltpu.VMEM((EK,), jnp.int32),
            pltpu.VMEM((EK, HP), jnp.float32),
            pltpu.VMEM((EK,), jnp.float32),
            pltpu.VMEM((N,), jnp.float32),
            pltpu.VMEM((N,), jnp.float32),
            pltpu.VMEM((16,), jnp.float32),
            pltpu.VMEM_SHARED((NP, HP), jnp.float32),
            pltpu.SemaphoreType.DMA,
            pltpu.SemaphoreType.DMA,
        ],
    )


_sc_edge = _make_sc_edge()


@jax.jit
def kernel(x, edge_index, W1, a_src1, a_dst1, b1, W2, a_src2, a_dst2, b2,
           Wout, bout):
    f32 = jnp.float32
    src = edge_index[0]
    dst = edge_index[1]
    zero_rows = jnp.zeros((NP, HP), f32)
    pad = jnp.zeros((D, HP - D), f32)

    w1p = jnp.concatenate([W1, pad], axis=1)
    hh1, as1, ad1, mx1 = _tc_embed(x, w1p, a_src1.reshape(D, 1),
                                   a_dst1.reshape(D, 1))
    s1 = mx1[0, 0] + mx1[0, 1]
    m1 = jnp.where(s1 >= 0, s1, 0.2 * s1)
    pm1 = _sc_edge(src, dst, hh1, as1.reshape(N), ad1.reshape(N),
                   jnp.full((16,), m1, f32), zero_rows)[:, :N]

    w2p = jnp.concatenate([W2, pad], axis=1)
    hh2, as2, ad2, mx2 = _tc_combine(pm1, hh1, as1, ad1, m1.reshape(1, 1),
                                     b1.reshape(1, D), w2p,
                                     a_src2.reshape(D, 1),
                                     a_dst2.reshape(D, 1))
    s2 = mx2[0, 0] + mx2[0, 1]
    m2 = jnp.where(s2 >= 0, s2, 0.2 * s2)
    pm2 = _sc_edge(src, dst, hh2, as2.reshape(N), ad2.reshape(N),
                   jnp.full((16,), m2, f32), zero_rows)[:, :N]

    (colsum,) = _tc_final(pm2, hh2, as2, ad2, m2.reshape(1, 1),
                          b2.reshape(1, D))
    pooled = colsum / jnp.float32(N)
    return pooled @ Wout + bout


# speedup vs baseline: 30.2342x; 1.4761x over previous
import functools

import jax
import jax.numpy as jnp
from jax import lax
from jax.experimental import pallas as pl
from jax.experimental.pallas import tpu as pltpu
from jax.experimental.pallas import tpu_sc as plsc

N = 10000
E = 320000
D = 128
HP = 144
BM = 1000
GM = N // BM
NC = 2
NS = 16
NTILES = NC * NS
TPW = E // NTILES
EK = 80
NCHUNK = TPW // EK
NP = 10240
RPT = NP // NS
NB = 3
NMAIN = (NCHUNK // NB) * NB


def _leaky(z):
    return jnp.where(z >= 0, z, 0.2 * z)


def _tc_embed_body(x_ref, w_ref, asv_ref, adv_ref,
                   hh_ref, as_ref, ad_ref, mx_ref):
    h = jnp.dot(x_ref[...], w_ref[...], preferred_element_type=jnp.float32)
    col = lax.broadcasted_iota(jnp.int32, h.shape, 1)
    a_s = jnp.dot(h[:, :D], asv_ref[...], preferred_element_type=jnp.float32)
    a_d = jnp.dot(h[:, :D], adv_ref[...], preferred_element_type=jnp.float32)
    hh_ref[...] = (h + jnp.where(col == D, 1.0, 0.0)
                   + a_s * jnp.where(col == D + 1, 1.0, 0.0))
    as_ref[...] = a_s
    ad_ref[...] = a_d
    mrow = jnp.concatenate([jnp.max(a_s, keepdims=True),
                            jnp.max(a_d, keepdims=True)], axis=1)

    @pl.when(pl.program_id(0) == 0)
    def _():
        mx_ref[...] = mrow

    @pl.when(pl.program_id(0) != 0)
    def _():
        mx_ref[...] = jnp.maximum(mx_ref[...], mrow)


def _tc_embed(x, wp, asv, adv):
    return pl.pallas_call(
        _tc_embed_body,
        grid=(GM,),
        in_specs=[
            pl.BlockSpec((BM, D), lambda i: (i, 0)),
            pl.BlockSpec((D, HP), lambda i: (0, 0)),
            pl.BlockSpec((D, 1), lambda i: (0, 0)),
            pl.BlockSpec((D, 1), lambda i: (0, 0)),
        ],
        out_specs=[
            pl.BlockSpec((BM, HP), lambda i: (i, 0)),
            pl.BlockSpec((BM, 1), lambda i: (i, 0)),
            pl.BlockSpec((BM, 1), lambda i: (i, 0)),
            pl.BlockSpec((1, 2), lambda i: (0, 0)),
        ],
        out_shape=[
            jax.ShapeDtypeStruct((N, HP), jnp.float32),
            jax.ShapeDtypeStruct((N, 1), jnp.float32),
            jax.ShapeDtypeStruct((N, 1), jnp.float32),
            jax.ShapeDtypeStruct((1, 2), jnp.float32),
        ],
        compiler_params=pltpu.CompilerParams(
            dimension_semantics=("arbitrary",)),
    )(x, wp, asv, adv)


def _combine(p_ref, hhp_ref, asp_ref, adp_ref, m_ref, b_ref):
    tot = p_ref[0] + p_ref[1]
    z = asp_ref[...] + adp_ref[...]
    w_self = jnp.exp(_leaky(z) - m_ref[0, 0])
    tot = tot + w_self * hhp_ref[...]
    return jnp.maximum(tot[:, :D] / (tot[:, D:D + 1] + 1e-16) + b_ref[...],
                       0.0)


def _tc_combine_body(p_ref, hhp_ref, asp_ref, adp_ref, m_ref, b_ref,
                     w_ref, asv_ref, adv_ref,
                     hh_ref, as_ref, ad_ref, mx_ref):
    xn = _combine(p_ref, hhp_ref, asp_ref, adp_ref, m_ref, b_ref)
    h = jnp.dot(xn, w_ref[...], preferred_element_type=jnp.float32)
    col = lax.broadcasted_iota(jnp.int32, h.shape, 1)
    a_s = jnp.dot(h[:, :D], asv_ref[...], preferred_element_type=jnp.float32)
    a_d = jnp.dot(h[:, :D], adv_ref[...], preferred_element_type=jnp.float32)
    hh_ref[...] = (h + jnp.where(col == D, 1.0, 0.0)
                   + a_s * jnp.where(col == D + 1, 1.0, 0.0))
    as_ref[...] = a_s
    ad_ref[...] = a_d
    mrow = jnp.concatenate([jnp.max(a_s, keepdims=True),
                            jnp.max(a_d, keepdims=True)], axis=1)

    @pl.when(pl.program_id(0) == 0)
    def _():
        mx_ref[...] = mrow

    @pl.when(pl.program_id(0) != 0)
    def _():
        mx_ref[...] = jnp.maximum(mx_ref[...], mrow)


def _tc_combine(pm, hhp, asp, adp, mscal, bias, wp, asv, adv):
    return pl.pallas_call(
        _tc_combine_body,
        grid=(GM,),
        in_specs=[
            pl.BlockSpec((2, BM, HP), lambda i: (0, i, 0)),
            pl.BlockSpec((BM, HP), lambda i: (i, 0)),
            pl.BlockSpec((BM, 1), lambda i: (i, 0)),
            pl.BlockSpec((BM, 1), lambda i: (i, 0)),
            pl.BlockSpec((1, 1), lambda i: (0, 0)),
            pl.BlockSpec((1, D), lambda i: (0, 0)),
            pl.BlockSpec((D, HP), lambda i: (0, 0)),
            pl.BlockSpec((D, 1), lambda i: (0, 0)),
            pl.BlockSpec((D, 1), lambda i: (0, 0)),
        ],
        out_specs=[
            pl.BlockSpec((BM, HP), lambda i: (i, 0)),
            pl.BlockSpec((BM, 1), lambda i: (i, 0)),
            pl.BlockSpec((BM, 1), lambda i: (i, 0)),
            pl.BlockSpec((1, 2), lambda i: (0, 0)),
        ],
        out_shape=[
            jax.ShapeDtypeStruct((N, HP), jnp.float32),
            jax.ShapeDtypeStruct((N, 1), jnp.float32),
            jax.ShapeDtypeStruct((N, 1), jnp.float32),
            jax.ShapeDtypeStruct((1, 2), jnp.float32),
        ],
        compiler_params=pltpu.CompilerParams(
            dimension_semantics=("arbitrary",)),
    )(pm, hhp, asp, adp, mscal, bias, wp, asv, adv)


def _tc_final_body(p_ref, hhp_ref, asp_ref, adp_ref, m_ref, b_ref, sum_ref):
    y = _combine(p_ref, hhp_ref, asp_ref, adp_ref, m_ref, b_ref)
    ps = jnp.sum(y, axis=0, keepdims=True)

    @pl.when(pl.program_id(0) == 0)
    def _():
        sum_ref[...] = ps

    @pl.when(pl.program_id(0) != 0)
    def _():
        sum_ref[...] = sum_ref[...] + ps


def _tc_final(pm, hhp, asp, adp, mscal, bias):
    return pl.pallas_call(
        _tc_final_body,
        grid=(GM,),
        in_specs=[
            pl.BlockSpec((2, BM, HP), lambda i: (0, i, 0)),
            pl.BlockSpec((BM, HP), lambda i: (i, 0)),
            pl.BlockSpec((BM, 1), lambda i: (i, 0)),
            pl.BlockSpec((BM, 1), lambda i: (i, 0)),
            pl.BlockSpec((1, 1), lambda i: (0, 0)),
            pl.BlockSpec((1, D), lambda i: (0, 0)),
        ],
        out_specs=[pl.BlockSpec((1, D), lambda i: (0, 0))],
        out_shape=[jax.ShapeDtypeStruct((1, D), jnp.float32)],
        compiler_params=pltpu.CompilerParams(
            dimension_semantics=("arbitrary",)),
    )(pm, hhp, asp, adp, mscal, bias)


def _sc_edge_body(src_hbm, dst_hbm, hh_hbm, ad_hbm, mv_hbm, zero_hbm,
                  out_hbm, *scr):
    srcs = scr[0:NB]
    dsts = scr[NB:2 * NB]
    rows = scr[2 * NB:3 * NB]
    ws = scr[3 * NB:4 * NB]
    adv = scr[4 * NB:5 * NB]
    m_l, msg_sh = scr[5 * NB:5 * NB + 2]
    gsem = scr[5 * NB + 2:6 * NB + 2]
    asem = scr[6 * NB + 2:7 * NB + 2]
    ssem = scr[7 * NB + 2:8 * NB + 2]

    c = lax.axis_index("c")
    s = lax.axis_index("s")
    wid = s * NC + c

    pltpu.sync_copy(mv_hbm, m_l)

    r0 = s * RPT
    pltpu.sync_copy(zero_hbm.at[pl.ds(r0, RPT)], msg_sh.at[pl.ds(r0, RPT)])
    plsc.subcore_barrier()

    base = wid * TPW
    m16 = m_l[...]
    lane = lax.iota(jnp.int32, 16)

    def start_chunk(ci, b):
        off = base + ci * EK
        pltpu.sync_copy(src_hbm.at[pl.ds(off, EK)], srcs[b])
        pltpu.sync_copy(dst_hbm.at[pl.ds(off, EK)], dsts[b])
        return (pltpu.async_copy(hh_hbm.at[srcs[b]], rows[b], gsem[b]),
                pltpu.async_copy(ad_hbm.at[dsts[b]], adv[b], asem[b]))

    def compute_chunk(b):
        rows_v, w_v, ad_v = rows[b], ws[b], adv[b]

        def wgrp(g, _):
            eid = g * 16 + lane
            a_s = plsc.load_gather(rows_v, [eid,
                                            jnp.full((16,), D + 1, jnp.int32)])
            z = a_s + ad_v[pl.ds(g * 16, 16)]
            w_v[pl.ds(g * 16, 16)] = jnp.exp(_leaky(z) - m16)
            return 0

        lax.fori_loop(0, EK // 16, wgrp, 0)

        def scale(j, _):
            wj = plsc.load_gather(w_v, [jnp.full((16,), j, jnp.int32)])
            for g in range(HP // 16):
                rows_v[j, pl.ds(g * 16, 16)] = (
                    rows_v[j, pl.ds(g * 16, 16)] * wj)
            return 0

        lax.fori_loop(0, EK, scale, 0)
        return pltpu.async_copy(rows_v, msg_sh.at[dsts[b]], ssem[b],
                                add=True)

    def group(gi, _):
        ci0 = gi * NB
        gd = [start_chunk(ci0 + b, b) for b in range(NB)]
        sd = []
        for b in range(NB):
            gd[b][0].wait()
            gd[b][1].wait()
            sd.append(compute_chunk(b))
        for b in range(NB):
            sd[b].wait()
        return 0

    lax.fori_loop(0, NMAIN // NB, group, 0)

    def tail(t, _):
        gd = start_chunk(NMAIN + t, 0)
        gd[0].wait()
        gd[1].wait()
        compute_chunk(0).wait()
        return 0

    lax.fori_loop(0, NCHUNK - NMAIN, tail, 0)
    plsc.subcore_barrier()
    pltpu.sync_copy(msg_sh.at[pl.ds(r0, RPT)], out_hbm.at[c, pl.ds(r0, RPT)])


def _make_sc_edge():
    mesh = plsc.VectorSubcoreMesh(core_axis_name="c", subcore_axis_name="s",
                                  num_cores=NC, num_subcores=NS)

    return pl.kernel(
        _sc_edge_body,
        out_type=jax.ShapeDtypeStruct((NC, NP, HP), jnp.float32),
        mesh=mesh,
        compiler_params=pltpu.CompilerParams(needs_layout_passes=False,
                                             use_tc_tiling_on_sc=False),
        scratch_types=(
            [pltpu.VMEM((EK,), jnp.int32) for _ in range(2 * NB)]
            + [pltpu.VMEM((EK, HP), jnp.float32) for _ in range(NB)]
            + [pltpu.VMEM((EK,), jnp.float32) for _ in range(2 * NB)]
            + [
                pltpu.VMEM((16,), jnp.float32),
                pltpu.VMEM_SHARED((NP, HP), jnp.float32),
            ]
            + [pltpu.SemaphoreType.DMA for _ in range(3 * NB)]
        ),
    )


_sc_edge = _make_sc_edge()


@jax.jit
def kernel(x, edge_index, W1, a_src1, a_dst1, b1, W2, a_src2, a_dst2, b2,
           Wout, bout):
    f32 = jnp.float32
    src = edge_index[0]
    dst = edge_index[1]
    zero_rows = jnp.zeros((NP, HP), f32)
    pad = jnp.zeros((D, HP - D), f32)

    w1p = jnp.concatenate([W1, pad], axis=1)
    hh1, as1, ad1, mx1 = _tc_embed(x, w1p, a_src1.reshape(D, 1),
                                   a_dst1.reshape(D, 1))
    s1 = mx1[0, 0] + mx1[0, 1]
    m1 = jnp.where(s1 >= 0, s1, 0.2 * s1)
    pm1 = _sc_edge(src, dst, hh1, ad1.reshape(N),
                   jnp.full((16,), m1, f32), zero_rows)[:, :N]

    w2p = jnp.concatenate([W2, pad], axis=1)
    hh2, as2, ad2, mx2 = _tc_combine(pm1, hh1, as1, ad1, m1.reshape(1, 1),
                                     b1.reshape(1, D), w2p,
                                     a_src2.reshape(D, 1),
                                     a_dst2.reshape(D, 1))
    s2 = mx2[0, 0] + mx2[0, 1]
    m2 = jnp.where(s2 >= 0, s2, 0.2 * s2)
    pm2 = _sc_edge(src, dst, hh2, ad2.reshape(N),
                   jnp.full((16,), m2, f32), zero_rows)[:, :N]

    (colsum,) = _tc_final(pm2, hh2, as2, ad2, m2.reshape(1, 1),
                          b2.reshape(1, D))
    pooled = colsum / jnp.float32(N)
    return pooled @ Wout + bout


# speedup vs baseline: 33.8829x; 1.1207x over previous
import functools

import jax
import jax.numpy as jnp
from jax import lax
from jax.experimental import pallas as pl
from jax.experimental.pallas import tpu as pltpu
from jax.experimental.pallas import tpu_sc as plsc

N = 10000
E = 320000
D = 128
HP = 144
BM = 1000
GM = N // BM
NC = 2
NS = 16
NTILES = NC * NS
TPW = E // NTILES
EK = 80
NCHUNK = TPW // EK
NP = 10240
RPT = NP // NS
NB = 3
NMAIN = (NCHUNK // NB) * NB


def _leaky(z):
    return jnp.where(z >= 0, z, 0.2 * z)


def _tc_embed_body(x_ref, w_ref, asv_ref, adv_ref,
                   hh_ref, as_ref, ad_ref, mx_ref):
    h = jnp.dot(x_ref[...], w_ref[...], preferred_element_type=jnp.float32)
    col = lax.broadcasted_iota(jnp.int32, h.shape, 1)
    a_s = jnp.dot(h[:, :D], asv_ref[...], preferred_element_type=jnp.float32)
    a_d = jnp.dot(h[:, :D], adv_ref[...], preferred_element_type=jnp.float32)
    hh_ref[...] = (h + jnp.where(col == D, 1.0, 0.0)
                   + a_s * jnp.where(col == D + 1, 1.0, 0.0))
    as_ref[...] = a_s
    ad_ref[...] = a_d
    mrow = jnp.concatenate([jnp.max(a_s, keepdims=True),
                            jnp.max(a_d, keepdims=True)], axis=1)

    @pl.when(pl.program_id(0) == 0)
    def _():
        mx_ref[...] = mrow

    @pl.when(pl.program_id(0) != 0)
    def _():
        mx_ref[...] = jnp.maximum(mx_ref[...], mrow)


def _tc_embed(x, wp, asv, adv):
    return pl.pallas_call(
        _tc_embed_body,
        grid=(GM,),
        in_specs=[
            pl.BlockSpec((BM, D), lambda i: (i, 0)),
            pl.BlockSpec((D, HP), lambda i: (0, 0)),
            pl.BlockSpec((D, 1), lambda i: (0, 0)),
            pl.BlockSpec((D, 1), lambda i: (0, 0)),
        ],
        out_specs=[
            pl.BlockSpec((BM, HP), lambda i: (i, 0)),
            pl.BlockSpec((BM, 1), lambda i: (i, 0)),
            pl.BlockSpec((BM, 1), lambda i: (i, 0)),
            pl.BlockSpec((1, 2), lambda i: (0, 0)),
        ],
        out_shape=[
            jax.ShapeDtypeStruct((N, HP), jnp.float32),
            jax.ShapeDtypeStruct((N, 1), jnp.float32),
            jax.ShapeDtypeStruct((N, 1), jnp.float32),
            jax.ShapeDtypeStruct((1, 2), jnp.float32),
        ],
        compiler_params=pltpu.CompilerParams(
            dimension_semantics=("arbitrary",)),
    )(x, wp, asv, adv)


def _combine(p_ref, hhp_ref, asp_ref, adp_ref, m_ref, b_ref):
    tot = p_ref[0] + p_ref[1]
    z = asp_ref[...] + adp_ref[...]
    w_self = jnp.exp(_leaky(z) - m_ref[0, 0])
    tot = tot + w_self * hhp_ref[...]
    return jnp.maximum(tot[:, :D] / (tot[:, D:D + 1] + 1e-16) + b_ref[...],
                       0.0)


def _tc_combine_body(p_ref, hhp_ref, asp_ref, adp_ref, m_ref, b_ref,
                     w_ref, asv_ref, adv_ref,
                     hh_ref, as_ref, ad_ref, mx_ref):
    xn = _combine(p_ref, hhp_ref, asp_ref, adp_ref, m_ref, b_ref)
    h = jnp.dot(xn, w_ref[...], preferred_element_type=jnp.float32)
    col = lax.broadcasted_iota(jnp.int32, h.shape, 1)
    a_s = jnp.dot(h[:, :D], asv_ref[...], preferred_element_type=jnp.float32)
    a_d = jnp.dot(h[:, :D], adv_ref[...], preferred_element_type=jnp.float32)
    hh_ref[...] = (h + jnp.where(col == D, 1.0, 0.0)
                   + a_s * jnp.where(col == D + 1, 1.0, 0.0))
    as_ref[...] = a_s
    ad_ref[...] = a_d
    mrow = jnp.concatenate([jnp.max(a_s, keepdims=True),
                            jnp.max(a_d, keepdims=True)], axis=1)

    @pl.when(pl.program_id(0) == 0)
    def _():
        mx_ref[...] = mrow

    @pl.when(pl.program_id(0) != 0)
    def _():
        mx_ref[...] = jnp.maximum(mx_ref[...], mrow)


def _tc_combine(pm, hhp, asp, adp, mscal, bias, wp, asv, adv):
    return pl.pallas_call(
        _tc_combine_body,
        grid=(GM,),
        in_specs=[
            pl.BlockSpec((2, BM, HP), lambda i: (0, i, 0)),
            pl.BlockSpec((BM, HP), lambda i: (i, 0)),
            pl.BlockSpec((BM, 1), lambda i: (i, 0)),
            pl.BlockSpec((BM, 1), lambda i: (i, 0)),
            pl.BlockSpec((1, 1), lambda i: (0, 0)),
            pl.BlockSpec((1, D), lambda i: (0, 0)),
            pl.BlockSpec((D, HP), lambda i: (0, 0)),
            pl.BlockSpec((D, 1), lambda i: (0, 0)),
            pl.BlockSpec((D, 1), lambda i: (0, 0)),
        ],
        out_specs=[
            pl.BlockSpec((BM, HP), lambda i: (i, 0)),
            pl.BlockSpec((BM, 1), lambda i: (i, 0)),
            pl.BlockSpec((BM, 1), lambda i: (i, 0)),
            pl.BlockSpec((1, 2), lambda i: (0, 0)),
        ],
        out_shape=[
            jax.ShapeDtypeStruct((N, HP), jnp.float32),
            jax.ShapeDtypeStruct((N, 1), jnp.float32),
            jax.ShapeDtypeStruct((N, 1), jnp.float32),
            jax.ShapeDtypeStruct((1, 2), jnp.float32),
        ],
        compiler_params=pltpu.CompilerParams(
            dimension_semantics=("arbitrary",)),
    )(pm, hhp, asp, adp, mscal, bias, wp, asv, adv)


def _tc_final_body(p_ref, hhp_ref, asp_ref, adp_ref, m_ref, b_ref, sum_ref):
    y = _combine(p_ref, hhp_ref, asp_ref, adp_ref, m_ref, b_ref)
    ps = jnp.sum(y, axis=0, keepdims=True)

    @pl.when(pl.program_id(0) == 0)
    def _():
        sum_ref[...] = ps

    @pl.when(pl.program_id(0) != 0)
    def _():
        sum_ref[...] = sum_ref[...] + ps


def _tc_final(pm, hhp, asp, adp, mscal, bias):
    return pl.pallas_call(
        _tc_final_body,
        grid=(GM,),
        in_specs=[
            pl.BlockSpec((2, BM, HP), lambda i: (0, i, 0)),
            pl.BlockSpec((BM, HP), lambda i: (i, 0)),
            pl.BlockSpec((BM, 1), lambda i: (i, 0)),
            pl.BlockSpec((BM, 1), lambda i: (i, 0)),
            pl.BlockSpec((1, 1), lambda i: (0, 0)),
            pl.BlockSpec((1, D), lambda i: (0, 0)),
        ],
        out_specs=[pl.BlockSpec((1, D), lambda i: (0, 0))],
        out_shape=[jax.ShapeDtypeStruct((1, D), jnp.float32)],
        compiler_params=pltpu.CompilerParams(
            dimension_semantics=("arbitrary",)),
    )(pm, hhp, asp, adp, mscal, bias)


def _sc_edge_body(src_hbm, dst_hbm, hh_hbm, ad_hbm, mv_hbm,
                  out_hbm, *scr):
    srcs = scr[0:NB]
    dsts = scr[NB:2 * NB]
    rows = scr[2 * NB:3 * NB]
    ws = scr[3 * NB:4 * NB]
    adv = scr[4 * NB:5 * NB]
    m_l, msg_sh = scr[5 * NB:5 * NB + 2]
    gsem = scr[5 * NB + 2:6 * NB + 2]
    asem = scr[6 * NB + 2:7 * NB + 2]
    ssem = scr[7 * NB + 2:8 * NB + 2]

    c = lax.axis_index("c")
    s = lax.axis_index("s")
    wid = s * NC + c

    pltpu.sync_copy(mv_hbm, m_l)

    r0 = s * RPT

    def zfill(j, _):
        for g in range(HP // 16):
            rows[0][j, pl.ds(g * 16, 16)] = jnp.zeros((16,), jnp.float32)
        return 0

    lax.fori_loop(0, EK, zfill, 0)
    for k in range(RPT // EK):
        pltpu.sync_copy(rows[0], msg_sh.at[pl.ds(r0 + k * EK, EK)])
    plsc.subcore_barrier()

    base = wid * TPW
    m16 = m_l[...]
    lane = lax.iota(jnp.int32, 16)

    def start_chunk(ci, b):
        off = base + ci * EK
        pltpu.sync_copy(src_hbm.at[pl.ds(off, EK)], srcs[b])
        pltpu.sync_copy(dst_hbm.at[pl.ds(off, EK)], dsts[b])
        return (pltpu.async_copy(hh_hbm.at[srcs[b]], rows[b], gsem[b]),
                pltpu.async_copy(ad_hbm.at[dsts[b]], adv[b], asem[b]))

    def compute_chunk(b):
        rows_v, w_v, ad_v = rows[b], ws[b], adv[b]

        def wgrp(g, _):
            eid = g * 16 + lane
            a_s = plsc.load_gather(rows_v, [eid,
                                            jnp.full((16,), D + 1, jnp.int32)])
            z = a_s + ad_v[pl.ds(g * 16, 16)]
            w_v[pl.ds(g * 16, 16)] = jnp.exp(_leaky(z) - m16)
            return 0

        lax.fori_loop(0, EK // 16, wgrp, 0)

        def scale(j, _):
            wj = plsc.load_gather(w_v, [jnp.full((16,), j, jnp.int32)])
            for g in range(HP // 16):
                rows_v[j, pl.ds(g * 16, 16)] = (
                    rows_v[j, pl.ds(g * 16, 16)] * wj)
            return 0

        lax.fori_loop(0, EK, scale, 0)
        return pltpu.async_copy(rows_v, msg_sh.at[dsts[b]], ssem[b],
                                add=True)

    def wait_gather(b):
        pltpu.make_async_copy(hh_hbm.at[srcs[b]], rows[b], gsem[b]).wait()
        pltpu.make_async_copy(ad_hbm.at[dsts[b]], adv[b], asem[b]).wait()

    def retire_and_refill(gi, b):
        pltpu.make_async_copy(rows[b], msg_sh.at[dsts[b]], ssem[b]).wait()

        @pl.when(gi < NMAIN // NB - 1)
        def _():
            start_chunk((gi + 1) * NB + b, b)

    for b in range(NB):
        start_chunk(b, b)

    def group(gi, _):
        for b in range(NB):
            wait_gather(b)
            compute_chunk(b)
            if b >= 1:
                retire_and_refill(gi, b - 1)
        retire_and_refill(gi, NB - 1)
        return 0

    lax.fori_loop(0, NMAIN // NB, group, 0)

    def tail(t, _):
        start_chunk(NMAIN + t, 0)
        wait_gather(0)
        compute_chunk(0)
        pltpu.make_async_copy(rows[0], msg_sh.at[dsts[0]], ssem[0]).wait()
        return 0

    lax.fori_loop(0, NCHUNK - NMAIN, tail, 0)
    plsc.subcore_barrier()
    pltpu.sync_copy(msg_sh.at[pl.ds(r0, RPT)], out_hbm.at[c, pl.ds(r0, RPT)])


def _make_sc_edge():
    mesh = plsc.VectorSubcoreMesh(core_axis_name="c", subcore_axis_name="s",
                                  num_cores=NC, num_subcores=NS)

    return pl.kernel(
        _sc_edge_body,
        out_type=jax.ShapeDtypeStruct((NC, NP, HP), jnp.float32),
        mesh=mesh,
        compiler_params=pltpu.CompilerParams(needs_layout_passes=False,
                                             use_tc_tiling_on_sc=False),
        scratch_types=(
            [pltpu.VMEM((EK,), jnp.int32) for _ in range(2 * NB)]
            + [pltpu.VMEM((EK, HP), jnp.float32) for _ in range(NB)]
            + [pltpu.VMEM((EK,), jnp.float32) for _ in range(2 * NB)]
            + [
                pltpu.VMEM((16,), jnp.float32),
                pltpu.VMEM_SHARED((NP, HP), jnp.float32),
            ]
            + [pltpu.SemaphoreType.DMA for _ in range(3 * NB)]
        ),
    )


_sc_edge = _make_sc_edge()


@jax.jit
def kernel(x, edge_index, W1, a_src1, a_dst1, b1, W2, a_src2, a_dst2, b2,
           Wout, bout):
    f32 = jnp.float32
    src = edge_index[0]
    dst = edge_index[1]
    pad = jnp.zeros((D, HP - D), f32)

    w1p = jnp.concatenate([W1, pad], axis=1)
    hh1, as1, ad1, mx1 = _tc_embed(x, w1p, a_src1.reshape(D, 1),
                                   a_dst1.reshape(D, 1))
    s1 = mx1[0, 0] + mx1[0, 1]
    m1 = jnp.where(s1 >= 0, s1, 0.2 * s1)
    pm1 = _sc_edge(src, dst, hh1, ad1.reshape(N),
                   jnp.full((16,), m1, f32))

    w2p = jnp.concatenate([W2, pad], axis=1)
    hh2, as2, ad2, mx2 = _tc_combine(pm1, hh1, as1, ad1, m1.reshape(1, 1),
                                     b1.reshape(1, D), w2p,
                                     a_src2.reshape(D, 1),
                                     a_dst2.reshape(D, 1))
    s2 = mx2[0, 0] + mx2[0, 1]
    m2 = jnp.where(s2 >= 0, s2, 0.2 * s2)
    pm2 = _sc_edge(src, dst, hh2, ad2.reshape(N),
                   jnp.full((16,), m2, f32))

    (colsum,) = _tc_final(pm2, hh2, as2, ad2, m2.reshape(1, 1),
                          b2.reshape(1, D))
    pooled = colsum / jnp.float32(N)
    return pooled @ Wout + bout


# speedup vs baseline: 34.1302x; 1.0073x over previous
import functools

import jax
import jax.numpy as jnp
from jax import lax
from jax.experimental import pallas as pl
from jax.experimental.pallas import tpu as pltpu
from jax.experimental.pallas import tpu_sc as plsc

N = 10000
E = 320000
D = 128
HP = 144
BM = 2000
GM = N // BM
NC = 2
NS = 16
NTILES = NC * NS
TPW = E // NTILES
EK = 80
NCHUNK = TPW // EK
NP = 10240
RPT = NP // NS
NB = 3
NMAIN = (NCHUNK // NB) * NB


def _leaky(z):
    return jnp.where(z >= 0, z, 0.2 * z)


def _tc_embed_body(x_ref, w_ref, asv_ref, adv_ref,
                   hh_ref, as_ref, ad_ref, mx_ref):
    h = jnp.dot(x_ref[...], w_ref[...], preferred_element_type=jnp.float32)
    col = lax.broadcasted_iota(jnp.int32, h.shape, 1)
    a_s = jnp.dot(h[:, :D], asv_ref[...], preferred_element_type=jnp.float32)
    a_d = jnp.dot(h[:, :D], adv_ref[...], preferred_element_type=jnp.float32)
    hh_ref[...] = (h + jnp.where(col == D, 1.0, 0.0)
                   + a_s * jnp.where(col == D + 1, 1.0, 0.0))
    as_ref[...] = a_s
    ad_ref[...] = a_d
    mrow = jnp.concatenate([jnp.max(a_s, keepdims=True),
                            jnp.max(a_d, keepdims=True)], axis=1)

    @pl.when(pl.program_id(0) == 0)
    def _():
        mx_ref[...] = mrow

    @pl.when(pl.program_id(0) != 0)
    def _():
        mx_ref[...] = jnp.maximum(mx_ref[...], mrow)


def _tc_embed(x, wp, asv, adv):
    return pl.pallas_call(
        _tc_embed_body,
        grid=(GM,),
        in_specs=[
            pl.BlockSpec((BM, D), lambda i: (i, 0)),
            pl.BlockSpec((D, HP), lambda i: (0, 0)),
            pl.BlockSpec((D, 1), lambda i: (0, 0)),
            pl.BlockSpec((D, 1), lambda i: (0, 0)),
        ],
        out_specs=[
            pl.BlockSpec((BM, HP), lambda i: (i, 0)),
            pl.BlockSpec((BM, 1), lambda i: (i, 0)),
            pl.BlockSpec((BM, 1), lambda i: (i, 0)),
            pl.BlockSpec((1, 2), lambda i: (0, 0)),
        ],
        out_shape=[
            jax.ShapeDtypeStruct((N, HP), jnp.float32),
            jax.ShapeDtypeStruct((N, 1), jnp.float32),
            jax.ShapeDtypeStruct((N, 1), jnp.float32),
            jax.ShapeDtypeStruct((1, 2), jnp.float32),
        ],
        compiler_params=pltpu.CompilerParams(
            dimension_semantics=("arbitrary",)),
    )(x, wp, asv, adv)


def _combine(p_ref, hhp_ref, asp_ref, adp_ref, m_ref, b_ref):
    tot = p_ref[0] + p_ref[1]
    z = asp_ref[...] + adp_ref[...]
    w_self = jnp.exp(_leaky(z) - m_ref[0, 0])
    tot = tot + w_self * hhp_ref[...]
    return jnp.maximum(tot[:, :D] / (tot[:, D:D + 1] + 1e-16) + b_ref[...],
                       0.0)


def _tc_combine_body(p_ref, hhp_ref, asp_ref, adp_ref, m_ref, b_ref,
                     w_ref, asv_ref, adv_ref,
                     hh_ref, as_ref, ad_ref, mx_ref):
    xn = _combine(p_ref, hhp_ref, asp_ref, adp_ref, m_ref, b_ref)
    h = jnp.dot(xn, w_ref[...], preferred_element_type=jnp.float32)
    col = lax.broadcasted_iota(jnp.int32, h.shape, 1)
    a_s = jnp.dot(h[:, :D], asv_ref[...], preferred_element_type=jnp.float32)
    a_d = jnp.dot(h[:, :D], adv_ref[...], preferred_element_type=jnp.float32)
    hh_ref[...] = (h + jnp.where(col == D, 1.0, 0.0)
                   + a_s * jnp.where(col == D + 1, 1.0, 0.0))
    as_ref[...] = a_s
    ad_ref[...] = a_d
    mrow = jnp.concatenate([jnp.max(a_s, keepdims=True),
                            jnp.max(a_d, keepdims=True)], axis=1)

    @pl.when(pl.program_id(0) == 0)
    def _():
        mx_ref[...] = mrow

    @pl.when(pl.program_id(0) != 0)
    def _():
        mx_ref[...] = jnp.maximum(mx_ref[...], mrow)


def _tc_combine(pm, hhp, asp, adp, mscal, bias, wp, asv, adv):
    return pl.pallas_call(
        _tc_combine_body,
        grid=(GM,),
        in_specs=[
            pl.BlockSpec((2, BM, HP), lambda i: (0, i, 0)),
            pl.BlockSpec((BM, HP), lambda i: (i, 0)),
            pl.BlockSpec((BM, 1), lambda i: (i, 0)),
            pl.BlockSpec((BM, 1), lambda i: (i, 0)),
            pl.BlockSpec((1, 1), lambda i: (0, 0)),
            pl.BlockSpec((1, D), lambda i: (0, 0)),
            pl.BlockSpec((D, HP), lambda i: (0, 0)),
            pl.BlockSpec((D, 1), lambda i: (0, 0)),
            pl.BlockSpec((D, 1), lambda i: (0, 0)),
        ],
        out_specs=[
            pl.BlockSpec((BM, HP), lambda i: (i, 0)),
            pl.BlockSpec((BM, 1), lambda i: (i, 0)),
            pl.BlockSpec((BM, 1), lambda i: (i, 0)),
            pl.BlockSpec((1, 2), lambda i: (0, 0)),
        ],
        out_shape=[
            jax.ShapeDtypeStruct((N, HP), jnp.float32),
            jax.ShapeDtypeStruct((N, 1), jnp.float32),
            jax.ShapeDtypeStruct((N, 1), jnp.float32),
            jax.ShapeDtypeStruct((1, 2), jnp.float32),
        ],
        compiler_params=pltpu.CompilerParams(
            dimension_semantics=("arbitrary",)),
    )(pm, hhp, asp, adp, mscal, bias, wp, asv, adv)


def _tc_final_body(p_ref, hhp_ref, asp_ref, adp_ref, m_ref, b_ref, sum_ref):
    y = _combine(p_ref, hhp_ref, asp_ref, adp_ref, m_ref, b_ref)
    ps = jnp.sum(y, axis=0, keepdims=True)

    @pl.when(pl.program_id(0) == 0)
    def _():
        sum_ref[...] = ps

    @pl.when(pl.program_id(0) != 0)
    def _():
        sum_ref[...] = sum_ref[...] + ps


def _tc_final(pm, hhp, asp, adp, mscal, bias):
    return pl.pallas_call(
        _tc_final_body,
        grid=(GM,),
        in_specs=[
            pl.BlockSpec((2, BM, HP), lambda i: (0, i, 0)),
            pl.BlockSpec((BM, HP), lambda i: (i, 0)),
            pl.BlockSpec((BM, 1), lambda i: (i, 0)),
            pl.BlockSpec((BM, 1), lambda i: (i, 0)),
            pl.BlockSpec((1, 1), lambda i: (0, 0)),
            pl.BlockSpec((1, D), lambda i: (0, 0)),
        ],
        out_specs=[pl.BlockSpec((1, D), lambda i: (0, 0))],
        out_shape=[jax.ShapeDtypeStruct((1, D), jnp.float32)],
        compiler_params=pltpu.CompilerParams(
            dimension_semantics=("arbitrary",)),
    )(pm, hhp, asp, adp, mscal, bias)


def _sc_edge_body(src_hbm, dst_hbm, hh_hbm, ad_hbm, mv_hbm,
                  out_hbm, *scr):
    srcs = scr[0:NB]
    dsts = scr[NB:2 * NB]
    rows = scr[2 * NB:3 * NB]
    ws = scr[3 * NB:4 * NB]
    adv = scr[4 * NB:5 * NB]
    m_l, msg_sh = scr[5 * NB:5 * NB + 2]
    gsem = scr[5 * NB + 2:6 * NB + 2]
    asem = scr[6 * NB + 2:7 * NB + 2]
    ssem = scr[7 * NB + 2:8 * NB + 2]

    c = lax.axis_index("c")
    s = lax.axis_index("s")
    wid = s * NC + c

    pltpu.sync_copy(mv_hbm, m_l)

    r0 = s * RPT

    def zfill(j, _):
        for g in range(HP // 16):
            rows[0][j, pl.ds(g * 16, 16)] = jnp.zeros((16,), jnp.float32)
        return 0

    lax.fori_loop(0, EK, zfill, 0)
    for k in range(RPT // EK):
        pltpu.sync_copy(rows[0], msg_sh.at[pl.ds(r0 + k * EK, EK)])
    plsc.subcore_barrier()

    base = wid * TPW
    m16 = m_l[...]
    lane = lax.iota(jnp.int32, 16)

    def start_chunk(ci, b):
        off = base + ci * EK
        pltpu.sync_copy(src_hbm.at[pl.ds(off, EK)], srcs[b])
        pltpu.sync_copy(dst_hbm.at[pl.ds(off, EK)], dsts[b])
        return (pltpu.async_copy(hh_hbm.at[srcs[b]], rows[b], gsem[b]),
                pltpu.async_copy(ad_hbm.at[dsts[b]], adv[b], asem[b]))

    def compute_chunk(b):
        rows_v, w_v, ad_v = rows[b], ws[b], adv[b]

        def wgrp(g, _):
            eid = g * 16 + lane
            a_s = plsc.load_gather(rows_v, [eid,
                                            jnp.full((16,), D + 1, jnp.int32)])
            z = a_s + ad_v[pl.ds(g * 16, 16)]
            w_v[pl.ds(g * 16, 16)] = jnp.exp(_leaky(z) - m16)
            return 0

        lax.fori_loop(0, EK // 16, wgrp, 0)

        def scale(j, _):
            wj = plsc.load_gather(w_v, [jnp.full((16,), j, jnp.int32)])
            for g in range(HP // 16):
                rows_v[j, pl.ds(g * 16, 16)] = (
                    rows_v[j, pl.ds(g * 16, 16)] * wj)
            return 0

        lax.fori_loop(0, EK, scale, 0)
        return pltpu.async_copy(rows_v, msg_sh.at[dsts[b]], ssem[b],
                                add=True)

    def wait_gather(b):
        pltpu.make_async_copy(hh_hbm.at[srcs[b]], rows[b], gsem[b]).wait()
        pltpu.make_async_copy(ad_hbm.at[dsts[b]], adv[b], asem[b]).wait()

    def retire_and_refill(gi, b):
        pltpu.make_async_copy(rows[b], msg_sh.at[dsts[b]], ssem[b]).wait()

        @pl.when(gi < NMAIN // NB - 1)
        def _():
            start_chunk((gi + 1) * NB + b, b)

    for b in range(NB):
        start_chunk(b, b)

    def group(gi, _):
        for b in range(NB):
            wait_gather(b)
            compute_chunk(b)
            if b >= 1:
                retire_and_refill(gi, b - 1)
        retire_and_refill(gi, NB - 1)
        return 0

    lax.fori_loop(0, NMAIN // NB, group, 0)

    def tail(t, _):
        start_chunk(NMAIN + t, 0)
        wait_gather(0)
        compute_chunk(0)
        pltpu.make_async_copy(rows[0], msg_sh.at[dsts[0]], ssem[0]).wait()
        return 0

    lax.fori_loop(0, NCHUNK - NMAIN, tail, 0)
    plsc.subcore_barrier()
    pltpu.sync_copy(msg_sh.at[pl.ds(r0, RPT)], out_hbm.at[c, pl.ds(r0, RPT)])


def _make_sc_edge():
    mesh = plsc.VectorSubcoreMesh(core_axis_name="c", subcore_axis_name="s",
                                  num_cores=NC, num_subcores=NS)

    return pl.kernel(
        _sc_edge_body,
        out_type=jax.ShapeDtypeStruct((NC, NP, HP), jnp.float32),
        mesh=mesh,
        compiler_params=pltpu.CompilerParams(needs_layout_passes=False,
                                             use_tc_tiling_on_sc=False),
        scratch_types=(
            [pltpu.VMEM((EK,), jnp.int32) for _ in range(2 * NB)]
            + [pltpu.VMEM((EK, HP), jnp.float32) for _ in range(NB)]
            + [pltpu.VMEM((EK,), jnp.float32) for _ in range(2 * NB)]
            + [
                pltpu.VMEM((16,), jnp.float32),
                pltpu.VMEM_SHARED((NP, HP), jnp.float32),
            ]
            + [pltpu.SemaphoreType.DMA for _ in range(3 * NB)]
        ),
    )


_sc_edge = _make_sc_edge()


@jax.jit
def kernel(x, edge_index, W1, a_src1, a_dst1, b1, W2, a_src2, a_dst2, b2,
           Wout, bout):
    f32 = jnp.float32
    src = edge_index[0]
    dst = edge_index[1]
    pad = jnp.zeros((D, HP - D), f32)

    w1p = jnp.concatenate([W1, pad], axis=1)
    hh1, as1, ad1, mx1 = _tc_embed(x, w1p, a_src1.reshape(D, 1),
                                   a_dst1.reshape(D, 1))
    s1 = mx1[0, 0] + mx1[0, 1]
    m1 = jnp.where(s1 >= 0, s1, 0.2 * s1)
    pm1 = _sc_edge(src, dst, hh1, ad1.reshape(N),
                   jnp.full((16,), m1, f32))

    w2p = jnp.concatenate([W2, pad], axis=1)
    hh2, as2, ad2, mx2 = _tc_combine(pm1, hh1, as1, ad1, m1.reshape(1, 1),
                                     b1.reshape(1, D), w2p,
                                     a_src2.reshape(D, 1),
                                     a_dst2.reshape(D, 1))
    s2 = mx2[0, 0] + mx2[0, 1]
    m2 = jnp.where(s2 >= 0, s2, 0.2 * s2)
    pm2 = _sc_edge(src, dst, hh2, ad2.reshape(N),
                   jnp.full((16,), m2, f32))

    (colsum,) = _tc_final(pm2, hh2, as2, ad2, m2.reshape(1, 1),
                          b2.reshape(1, D))
    pooled = colsum / jnp.float32(N)
    return pooled @ Wout + bout
